# baseline (speedup 1.0000x reference)
import functools

import jax
import jax.numpy as jnp
from jax import lax
from jax.experimental import pallas as pl
from jax.experimental.pallas import tpu as pltpu
from jax.experimental.pallas import tpu_sc as plsc

H = 64
VPAD = 64


def _proj_body(table_ref, w_ref, b_ref, out_ref):
    out_ref[...] = (
        lax.dot_general(
            table_ref[...], w_ref[...],
            (((1,), (1,)), ((), ())),
            preferred_element_type=jnp.float32,
        )
        + b_ref[...]
    )


@functools.partial(jax.jit, static_argnums=(2, 3))
def _gather_call(proj, ids, b_per_w, chunk):
    mesh = plsc.VectorSubcoreMesh(core_axis_name="c", subcore_axis_name="s")
    num_chunks = b_per_w // chunk
    B = ids.shape[0]

    @functools.partial(
        pl.kernel,
        mesh=mesh,
        out_type=jax.ShapeDtypeStruct((B, H), jnp.float32),
        scratch_types=[
            pltpu.VMEM((b_per_w,), jnp.int32),
            pltpu.VMEM((chunk, H), jnp.float32),
            pltpu.VMEM((chunk, H), jnp.float32),
            pltpu.SemaphoreType.DMA,
            pltpu.SemaphoreType.DMA,
            pltpu.SemaphoreType.DMA,
            pltpu.SemaphoreType.DMA,
        ],
        compiler_params=pltpu.CompilerParams(use_tc_tiling_on_sc=False),
    )
    def k(proj_hbm, idx_hbm, out_hbm, idx_all, rows0, rows1, sg0, sg1, ss0, ss1):
        wid = lax.axis_index("s") * 2 + lax.axis_index("c")
        base = wid * b_per_w
        rows = (rows0, rows1)
        sg = (sg0, sg1)
        ss = (ss0, ss1)

        pltpu.sync_copy(idx_hbm.at[pl.ds(base, b_per_w)], idx_all)

        def start_gather(g, b):
            return pltpu.async_copy(
                proj_hbm.at[idx_all.at[pl.ds(g * chunk, chunk)]], rows[b], sg[b]
            )

        def start_store(g, b):
            return pltpu.async_copy(
                rows[b], out_hbm.at[pl.ds(base + g * chunk, chunk)], ss[b]
            )

        gathers = {0: start_gather(0, 0)}
        stores = {}
        for g in range(num_chunks):
            b = g & 1
            gathers[g].wait()
            if g + 1 < num_chunks:
                if g >= 1:
                    stores[g - 1].wait()
                gathers[g + 1] = start_gather(g + 1, b ^ 1)
            stores[g] = start_store(g, b)
        stores[num_chunks - 1].wait()
        if num_chunks >= 2:
            stores[num_chunks - 2].wait()

    return k(proj, ids)


@jax.jit
def kernel(week_ids, table, W, b):
    Bseq, L = week_ids.shape
    ids = week_ids.reshape(-1).astype(jnp.int32)

    table_pad = jnp.zeros((VPAD, H), jnp.float32).at[: table.shape[0]].set(table)
    proj = pl.pallas_call(
        _proj_body,
        out_shape=jax.ShapeDtypeStruct((VPAD, H), jnp.float32),
    )(table_pad, W, b.reshape(1, H))

    B = ids.shape[0]
    b_per_w = B // 32
    out = _gather_call(proj, ids, b_per_w, 640)
    return out.reshape(Bseq, L, H)


# speedup vs baseline: 2.3120x; 2.3120x over previous
import functools

import jax
import jax.numpy as jnp
from jax import lax
from jax.experimental import pallas as pl
from jax.experimental.pallas import tpu as pltpu
from jax.experimental.pallas import tpu_sc as plsc

H = 64
VPAD = 64


def _proj_body(table_ref, w_ref, b_ref, out_ref):
    out_ref[...] = (
        lax.dot_general(
            table_ref[...], w_ref[...],
            (((1,), (1,)), ((), ())),
            preferred_element_type=jnp.float32,
        )
        + b_ref[...]
    )


@functools.partial(jax.jit, static_argnums=(2, 3))
def _gather_call(proj, ids, b_per_w, chunk):
    mesh = plsc.VectorSubcoreMesh(core_axis_name="c", subcore_axis_name="s")
    num_chunks = b_per_w // chunk
    B = ids.shape[0]

    @functools.partial(
        pl.kernel,
        mesh=mesh,
        out_type=jax.ShapeDtypeStruct((B, H), jnp.float32),
        scratch_types=[
            pltpu.VMEM((b_per_w,), jnp.int32),
            pltpu.VMEM((chunk, H), jnp.float32),
            pltpu.VMEM((chunk, H), jnp.float32),
            pltpu.VMEM_SHARED((VPAD, H), jnp.float32),
            pltpu.SemaphoreType.DMA,
            pltpu.SemaphoreType.DMA,
            pltpu.SemaphoreType.DMA,
            pltpu.SemaphoreType.DMA,
        ],
        compiler_params=pltpu.CompilerParams(use_tc_tiling_on_sc=False),
    )
    def k(proj_hbm, idx_hbm, out_hbm, idx_all, rows0, rows1, proj_sp, sg0, sg1, ss0, ss1):
        wid = lax.axis_index("s") * 2 + lax.axis_index("c")
        base = wid * b_per_w
        rows = (rows0, rows1)
        sg = (sg0, sg1)
        ss = (ss0, ss1)

        @pl.when(lax.axis_index("s") == 0)
        def _():
            pltpu.sync_copy(proj_hbm, proj_sp)

        pltpu.sync_copy(idx_hbm.at[pl.ds(base, b_per_w)], idx_all)
        plsc.subcore_barrier()

        def start_gather(g, b):
            return pltpu.async_copy(
                proj_sp.at[idx_all.at[pl.ds(g * chunk, chunk)]], rows[b], sg[b]
            )

        def start_store(g, b):
            return pltpu.async_copy(
                rows[b], out_hbm.at[pl.ds(base + g * chunk, chunk)], ss[b]
            )

        gathers = {0: start_gather(0, 0)}
        stores = {}
        for g in range(num_chunks):
            b = g & 1
            gathers[g].wait()
            if g + 1 < num_chunks:
                if g >= 1:
                    stores[g - 1].wait()
                gathers[g + 1] = start_gather(g + 1, b ^ 1)
            stores[g] = start_store(g, b)
        stores[num_chunks - 1].wait()
        if num_chunks >= 2:
            stores[num_chunks - 2].wait()

    return k(proj, ids)


@jax.jit
def kernel(week_ids, table, W, b):
    Bseq, L = week_ids.shape
    ids = week_ids.reshape(-1).astype(jnp.int32)

    table_pad = jnp.zeros((VPAD, H), jnp.float32).at[: table.shape[0]].set(table)
    proj = pl.pallas_call(
        _proj_body,
        out_shape=jax.ShapeDtypeStruct((VPAD, H), jnp.float32),
    )(table_pad, W, b.reshape(1, H))

    B = ids.shape[0]
    b_per_w = B // 32
    out = _gather_call(proj, ids, b_per_w, 640)
    return out.reshape(Bseq, L, H)
